# baseline (speedup 1.0000x reference)
import functools

import jax
import jax.numpy as jnp
from jax import lax
from jax.experimental import pallas as pl
from jax.experimental.pallas import tpu as pltpu
from jax.experimental.pallas import tpu_sc as plsc

_INDEX = 2
_ROWS, _COLS = 4096, 2048


def _make_sc_copy():
    info = plsc.get_sparse_core_info()
    nc, ns = info.num_cores, info.num_subcores
    nw = nc * ns
    rows_per = _ROWS // nw

    mesh = plsc.VectorSubcoreMesh(core_axis_name="c", subcore_axis_name="s")

    @functools.partial(
        pl.kernel,
        mesh=mesh,
        out_type=jax.ShapeDtypeStruct((_ROWS, _COLS), jnp.float32),
    )
    def sc_copy(x_hbm, out_hbm):
        wid = lax.axis_index("s") * nc + lax.axis_index("c")
        base = wid * rows_per
        pltpu.sync_copy(
            x_hbm.at[_INDEX, pl.ds(base, rows_per)],
            out_hbm.at[pl.ds(base, rows_per)],
        )

    return sc_copy


_sc_copy = _make_sc_copy()


def kernel(x):
    return _sc_copy(x)


# speedup vs baseline: 1.0181x; 1.0181x over previous
import jax
import jax.numpy as jnp
from jax.experimental import pallas as pl
from jax.experimental.pallas import tpu as pltpu

_INDEX = 2
_ROWS, _COLS = 4096, 2048


def _body(x_hbm, o_hbm, sem):
    copy = pltpu.make_async_copy(x_hbm.at[_INDEX], o_hbm, sem)
    copy.start()
    copy.wait()


def kernel(x):
    return pl.pallas_call(
        _body,
        in_specs=[pl.BlockSpec(memory_space=pl.ANY)],
        out_specs=pl.BlockSpec(memory_space=pl.ANY),
        out_shape=jax.ShapeDtypeStruct((_ROWS, _COLS), jnp.float32),
        scratch_shapes=[pltpu.SemaphoreType.DMA],
    )(x)


# speedup vs baseline: 47.8336x; 46.9820x over previous
import jax
import jax.numpy as jnp
from jax.experimental import pallas as pl
from jax.experimental.pallas import tpu as pltpu

_INDEX = 2
_ROWS, _COLS = 4096, 2048
_CHUNK = 256
_NCHUNKS = _ROWS // _CHUNK


def _body(x_hbm, o_hbm, buf, in_sems, out_sems):
    def in_copy(i):
        return pltpu.make_async_copy(
            x_hbm.at[_INDEX, pl.ds(i * _CHUNK, _CHUNK)],
            buf.at[pl.ds(i * _CHUNK, _CHUNK)],
            in_sems.at[i],
        )

    def out_copy(i):
        return pltpu.make_async_copy(
            buf.at[pl.ds(i * _CHUNK, _CHUNK)],
            o_hbm.at[pl.ds(i * _CHUNK, _CHUNK)],
            out_sems.at[i],
        )

    for i in range(_NCHUNKS):
        in_copy(i).start()
    for i in range(_NCHUNKS):
        in_copy(i).wait()
        out_copy(i).start()
    for i in range(_NCHUNKS):
        out_copy(i).wait()


def kernel(x):
    return pl.pallas_call(
        _body,
        in_specs=[pl.BlockSpec(memory_space=pl.ANY)],
        out_specs=pl.BlockSpec(memory_space=pl.ANY),
        out_shape=jax.ShapeDtypeStruct((_ROWS, _COLS), jnp.float32),
        scratch_shapes=[
            pltpu.VMEM((_ROWS, _COLS), jnp.float32),
            pltpu.SemaphoreType.DMA((_NCHUNKS,)),
            pltpu.SemaphoreType.DMA((_NCHUNKS,)),
        ],
    )(x)
